# baseline (speedup 1.0000x reference)
import jax
import jax.numpy as jnp
from jax import lax
from jax.experimental import pallas as pl
from jax.experimental.pallas import tpu as pltpu
from jax.experimental.pallas import tpu_sc as plsc

VOCAB = 1000000
D = 64
SEQ = 200
BATCH = 4096

NC = 2
NS = 16
NW = NC * NS
SEQS_PER_W = BATCH // NW
CHUNK = SEQ
NBUF = 4
SPLIT = 128
REST = CHUNK - SPLIT


def _positional_encoding_table():
    pos = jnp.arange(SEQ, dtype=jnp.float32)[:, None]
    i = jnp.arange(0, D, 2, dtype=jnp.float32)
    div = jnp.exp(-jnp.log(10000.0) * i / D)
    ang = pos * div[None, :]
    pe = jnp.zeros((SEQ, D), dtype=jnp.float32)
    pe = pe.at[:, 0::2].set(jnp.sin(ang))
    pe = pe.at[:, 1::2].set(jnp.cos(ang))
    return pe


def _sc_body(table_hbm, xt_hbm, pe_hbm, out_hbm,
             xts_v, idx_v, pe_v, bufs, gsems, ssems):
    wid = lax.axis_index("s") * NC + lax.axis_index("c")
    seq_base = wid * SEQS_PER_W

    pltpu.sync_copy(xt_hbm.at[:, pl.ds(seq_base, SEQS_PER_W)], xts_v)
    pltpu.sync_copy(pe_hbm, pe_v)

    iota16 = lax.iota(jnp.int32, 16)
    starts = list(range(0, SEQ - 15, 16)) + [SEQ - 16]

    @pl.loop(0, SEQS_PER_W)
    def _tr(c):
        cols = jnp.broadcast_to(c, (16,)).astype(jnp.int32)
        for s0 in starts:
            rows = iota16 + s0
            idx_v[c, pl.ds(s0, 16)] = plsc.load_gather(xts_v, [rows, cols])

    def start_gather(c, b):
        pltpu.async_copy(table_hbm.at[idx_v.at[c, pl.ds(0, SPLIT)]],
                         bufs.at[b, pl.ds(0, SPLIT)], gsems.at[b])
        pltpu.async_copy(table_hbm.at[idx_v.at[c, pl.ds(SPLIT, REST)]],
                         bufs.at[b, pl.ds(SPLIT, REST)], gsems.at[b])

    def wait_gather(b):
        pltpu.make_async_copy(table_hbm.at[idx_v.at[0, pl.ds(0, SPLIT)]],
                              bufs.at[b, pl.ds(0, SPLIT)], gsems.at[b]).wait()
        pltpu.make_async_copy(table_hbm.at[idx_v.at[0, pl.ds(0, REST)]],
                              bufs.at[b, pl.ds(SPLIT, REST)], gsems.at[b]).wait()

    def drain_store(b):
        pltpu.make_async_copy(bufs.at[b], out_hbm.at[0], ssems.at[b]).wait()

    for b in range(NBUF - 1):
        start_gather(b, b)

    @pl.loop(0, SEQS_PER_W // NBUF)
    def _grp(g):
        c0 = g * NBUF
        for b in range(NBUF):
            c = c0 + b
            bb = (b + NBUF - 1) % NBUF

            @pl.when(c + NBUF - 1 < SEQS_PER_W)
            def _():
                @pl.when(c >= 1)
                def _():
                    drain_store(bb)
                start_gather(c + NBUF - 1, bb)

            wait_gather(b)

            @pl.loop(0, CHUNK, unroll=8)
            def _row(r):
                for j in range(D // 16):
                    sl = pl.ds(j * 16, 16)
                    plsc.addupdate(bufs.at[b, r, sl], pe_v[r, sl])

            pltpu.async_copy(bufs.at[b], out_hbm.at[seq_base + c],
                             ssems.at[b])

    for b in range(NBUF):
        drain_store(b)


@jax.jit
def _embed(x, token_emb, pe):
    xt = x.astype(jnp.int32).T
    mesh = plsc.VectorSubcoreMesh(core_axis_name="c", subcore_axis_name="s")
    out = pl.kernel(
        _sc_body,
        out_type=jax.ShapeDtypeStruct((BATCH, SEQ, D), jnp.float32),
        mesh=mesh,
        compiler_params=pltpu.CompilerParams(use_tc_tiling_on_sc=False,
                                             needs_layout_passes=False),
        scratch_types=[
            pltpu.VMEM((SEQ, SEQS_PER_W), jnp.int32),
            pltpu.VMEM((SEQS_PER_W, SEQ), jnp.int32),
            pltpu.VMEM((SEQ, D), jnp.float32),
            pltpu.VMEM((NBUF, CHUNK, D), jnp.float32),
            pltpu.SemaphoreType.DMA((NBUF,)),
            pltpu.SemaphoreType.DMA((NBUF,)),
        ],
    )(token_emb, xt, pe)
    return out


def kernel(x, token_emb):
    pe = _positional_encoding_table()
    return _embed(x, token_emb, pe)


# speedup vs baseline: 1.0782x; 1.0782x over previous
import jax
import jax.numpy as jnp
from jax import lax
from jax.experimental import pallas as pl
from jax.experimental.pallas import tpu as pltpu
from jax.experimental.pallas import tpu_sc as plsc

VOCAB = 1000000
D = 64
SEQ = 200
BATCH = 4096

NC = 2
NS = 16
NW = NC * NS
TOTAL_ROWS = BATCH * SEQ
BSZ = 256
BLOCKS_PER_S = BATCH // BSZ
CHUNKS_PER_W = SEQ * BLOCKS_PER_S // NW
ROWS_PER_W = CHUNKS_PER_W * BSZ
NBUF = 4
SPLIT = 128


def _positional_encoding_table():
    pos = jnp.arange(SEQ, dtype=jnp.float32)[:, None]
    i = jnp.arange(0, D, 2, dtype=jnp.float32)
    div = jnp.exp(-jnp.log(10000.0) * i / D)
    ang = pos * div[None, :]
    pe = jnp.zeros((SEQ, D), dtype=jnp.float32)
    pe = pe.at[:, 0::2].set(jnp.sin(ang))
    pe = pe.at[:, 1::2].set(jnp.cos(ang))
    return pe


def _sc_body(table_hbm, idx_hbm, pe_hbm, out_hbm,
             idx_v, pe_v, bufs, gsems, ssems):
    wid = lax.axis_index("s") * NC + lax.axis_index("c")
    base = wid * ROWS_PER_W
    chunk_base = wid * CHUNKS_PER_W

    pltpu.sync_copy(idx_hbm.at[pl.ds(base, ROWS_PER_W)], idx_v)
    pltpu.sync_copy(pe_hbm, pe_v)

    def start_gather(c, b):
        for h in range(BSZ // SPLIT):
            pltpu.async_copy(
                table_hbm.at[idx_v.at[pl.ds(c * BSZ + h * SPLIT, SPLIT)]],
                bufs.at[b, pl.ds(h * SPLIT, SPLIT)], gsems.at[b])

    def wait_gather(b):
        for h in range(BSZ // SPLIT):
            pltpu.make_async_copy(
                table_hbm.at[idx_v.at[pl.ds(0, SPLIT)]],
                bufs.at[b, pl.ds(h * SPLIT, SPLIT)], gsems.at[b]).wait()

    def drain_store(b):
        pltpu.make_async_copy(bufs.at[b], out_hbm.at[pl.ds(0, BSZ)],
                              ssems.at[b]).wait()

    for b in range(NBUF - 1):
        start_gather(b, b)

    @pl.loop(0, CHUNKS_PER_W // NBUF)
    def _grp(g):
        c0 = g * NBUF
        for b in range(NBUF):
            c = c0 + b
            bb = (b + NBUF - 1) % NBUF

            @pl.when(c + NBUF - 1 < CHUNKS_PER_W)
            def _():
                @pl.when(c >= 1)
                def _():
                    drain_store(bb)
                start_gather(c + NBUF - 1, bb)

            wait_gather(b)

            s = (chunk_base + c) // BLOCKS_PER_S
            pe_regs = [pe_v[s, pl.ds(j * 16, 16)] for j in range(D // 16)]

            @pl.loop(0, BSZ, unroll=8)
            def _row(r):
                for j in range(D // 16):
                    plsc.addupdate(bufs.at[b, r, pl.ds(j * 16, 16)],
                                   pe_regs[j])

            pltpu.async_copy(bufs.at[b],
                             out_hbm.at[pl.ds(base + c * BSZ, BSZ)],
                             ssems.at[b])

    for b in range(NBUF):
        drain_store(b)


@jax.jit
def _embed(x, token_emb, pe):
    xt = x.astype(jnp.int32).T.reshape(TOTAL_ROWS)
    mesh = plsc.VectorSubcoreMesh(core_axis_name="c", subcore_axis_name="s")
    out = pl.kernel(
        _sc_body,
        out_type=jax.ShapeDtypeStruct((TOTAL_ROWS, D), jnp.float32),
        mesh=mesh,
        compiler_params=pltpu.CompilerParams(use_tc_tiling_on_sc=False,
                                             needs_layout_passes=False),
        scratch_types=[
            pltpu.VMEM((ROWS_PER_W,), jnp.int32),
            pltpu.VMEM((SEQ, D), jnp.float32),
            pltpu.VMEM((NBUF, BSZ, D), jnp.float32),
            pltpu.SemaphoreType.DMA((NBUF,)),
            pltpu.SemaphoreType.DMA((NBUF,)),
        ],
    )(token_emb, xt, pe)
    return out.reshape(SEQ, BATCH, D).transpose(1, 0, 2)


def kernel(x, token_emb):
    pe = _positional_encoding_table()
    return _embed(x, token_emb, pe)


# speedup vs baseline: 1.0802x; 1.0019x over previous
import jax
import jax.numpy as jnp
from jax import lax
from jax.experimental import pallas as pl
from jax.experimental.pallas import tpu as pltpu
from jax.experimental.pallas import tpu_sc as plsc

VOCAB = 1000000
D = 64
SEQ = 200
BATCH = 4096

NC = 2
NS = 16
NW = NC * NS
TOTAL_ROWS = BATCH * SEQ
BSZ = 128
TILE_COLS = BATCH // 128
CHUNKS_PER_W = TOTAL_ROWS // BSZ // NW
ROWS_PER_W = CHUNKS_PER_W * BSZ
NBUF = 4


def _positional_encoding_table():
    pos = jnp.arange(SEQ, dtype=jnp.float32)[:, None]
    i = jnp.arange(0, D, 2, dtype=jnp.float32)
    div = jnp.exp(-jnp.log(10000.0) * i / D)
    ang = pos * div[None, :]
    pe = jnp.zeros((SEQ, D), dtype=jnp.float32)
    pe = pe.at[:, 0::2].set(jnp.sin(ang))
    pe = pe.at[:, 1::2].set(jnp.cos(ang))
    return pe


def _sc_body(table_hbm, idx_hbm, pe_hbm, out_hbm,
             idx_v, pe_v, bufs, gsems, ssems):
    wid = lax.axis_index("s") * NC + lax.axis_index("c")
    base = wid * ROWS_PER_W
    chunk_base = wid * CHUNKS_PER_W

    pltpu.sync_copy(idx_hbm.at[pl.ds(base, ROWS_PER_W)], idx_v)
    pltpu.sync_copy(pe_hbm, pe_v)

    def start_gather(c, b):
        pltpu.async_copy(table_hbm.at[idx_v.at[pl.ds(c * BSZ, BSZ)]],
                         bufs.at[b], gsems.at[b])

    def wait_gather(b):
        pltpu.make_async_copy(table_hbm.at[idx_v.at[pl.ds(0, BSZ)]],
                              bufs.at[b], gsems.at[b]).wait()

    def drain_store(b):
        pltpu.make_async_copy(bufs.at[b], out_hbm.at[pl.ds(0, BSZ)],
                              ssems.at[b]).wait()

    for b in range(NBUF - 1):
        start_gather(b, b)

    @pl.loop(0, CHUNKS_PER_W // NBUF)
    def _grp(g):
        c0 = g * NBUF
        for b in range(NBUF):
            c = c0 + b
            bb = (b + NBUF - 1) % NBUF

            @pl.when(c + NBUF - 1 < CHUNKS_PER_W)
            def _():
                @pl.when(c >= 1)
                def _():
                    drain_store(bb)
                start_gather(c + NBUF - 1, bb)

            wait_gather(b)

            rr = chunk_base + c
            t = rr // 8
            s = 8 * (t // TILE_COLS) + (rr % 8)
            orow = s * BATCH + (t % TILE_COLS) * BSZ

            pe_regs = [pe_v[s, pl.ds(j * 16, 16)] for j in range(D // 16)]

            @pl.loop(0, BSZ, unroll=8)
            def _row(r):
                for j in range(D // 16):
                    plsc.addupdate(bufs.at[b, r, pl.ds(j * 16, 16)],
                                   pe_regs[j])

            pltpu.async_copy(bufs.at[b], out_hbm.at[pl.ds(orow, BSZ)],
                             ssems.at[b])

    for b in range(NBUF):
        drain_store(b)


@jax.jit
def _embed(x, token_emb, pe):
    xq = (x.astype(jnp.int32).T
          .reshape(SEQ // 8, 8, BATCH // 128, 128)
          .transpose(0, 2, 1, 3))
    xt = xq.reshape(TOTAL_ROWS)
    mesh = plsc.VectorSubcoreMesh(core_axis_name="c", subcore_axis_name="s")
    out = pl.kernel(
        _sc_body,
        out_type=jax.ShapeDtypeStruct((TOTAL_ROWS, D), jnp.float32),
        mesh=mesh,
        compiler_params=pltpu.CompilerParams(use_tc_tiling_on_sc=False,
                                             needs_layout_passes=False),
        scratch_types=[
            pltpu.VMEM((ROWS_PER_W,), jnp.int32),
            pltpu.VMEM((SEQ, D), jnp.float32),
            pltpu.VMEM((NBUF, BSZ, D), jnp.float32),
            pltpu.SemaphoreType.DMA((NBUF,)),
            pltpu.SemaphoreType.DMA((NBUF,)),
        ],
    )(token_emb, xt, pe)
    return out.reshape(SEQ, BATCH, D).transpose(1, 0, 2)


def kernel(x, token_emb):
    pe = _positional_encoding_table()
    return _embed(x, token_emb, pe)
